# baseline (speedup 1.0000x reference)
import jax
import jax.numpy as jnp
from jax import lax
from jax.experimental import pallas as pl
from jax.experimental.pallas import tpu as pltpu
from jax.experimental.pallas import tpu_sc as plsc

NUM_USERS = 5000
N_NODES = 10000
D = 128
DB = D // 16

NC = 2
NS = 16
NW = NC * NS

E_TOTAL = 320000
CH = 80
EDGES_PER_TILE = E_TOTAL // NW
N_CHUNKS = EDGES_PER_TILE // CH

N_RCH = N_NODES // CH
N_RCH_PER_TILE = (N_RCH + NS - 1) // NS

B = 16384
B_PER_TILE = B // NW
BCH = 64
NBCH = B_PER_TILE // BCH

_mesh = plsc.VectorSubcoreMesh(
    core_axis_name="c", subcore_axis_name="s", num_cores=NC, num_subcores=NS
)

_f32 = jnp.float32
_i32 = jnp.int32


def _splat(x):
    return jnp.full((16,), x, _i32)


def _propagate_body(full_h, row_h, col_h, val_h, p0_h, p1_h,
                    acc, colbuf, rowbuf, valbuf, inb, outb,
                    lsem, gsem, ssem):
    c = lax.axis_index("c")
    s = lax.axis_index("s")
    wid = c * NS + s
    ebase = wid * EDGES_PER_TILE

    @pl.when(c == 1)
    def _zero_staging():
        @plsc.parallel_loop(0, CH, unroll=4)
        def _z(r):
            for j in range(DB):
                inb[0, r, pl.ds(16 * j, 16)] = jnp.zeros((16,), _f32)

    def init_chunk(k, carry):
        idx = s + k * NS

        @pl.when(idx < N_RCH)
        def _():
            base = idx * CH

            @pl.when(c == 0)
            def _c0():
                pltpu.sync_copy(full_h.at[pl.ds(base, CH)],
                                acc.at[pl.ds(base, CH)])

            @pl.when(c == 1)
            def _c1():
                pltpu.sync_copy(inb.at[0], acc.at[pl.ds(base, CH)])

        return carry

    lax.fori_loop(0, N_RCH_PER_TILE, init_chunk, 0)
    plsc.subcore_barrier()

    def edge_slice(k):
        return pl.ds(ebase + k * CH, CH)

    def start_idx(k, b4, bb):
        pltpu.async_copy(col_h.at[edge_slice(k)], colbuf.at[b4], lsem.at[bb])
        pltpu.async_copy(row_h.at[edge_slice(k)], rowbuf.at[b4], lsem.at[bb])
        pltpu.async_copy(val_h.at[edge_slice(k)], valbuf.at[b4], lsem.at[bb])

    def wait_idx(k, b4, bb):
        pltpu.make_async_copy(col_h.at[edge_slice(k)], colbuf.at[b4],
                              lsem.at[bb]).wait()
        pltpu.make_async_copy(row_h.at[edge_slice(k)], rowbuf.at[b4],
                              lsem.at[bb]).wait()
        pltpu.make_async_copy(val_h.at[edge_slice(k)], valbuf.at[b4],
                              lsem.at[bb]).wait()

    def start_g(b4, bb):
        pltpu.async_copy(full_h.at[colbuf.at[b4]], inb.at[bb], gsem.at[bb])

    def wait_g(b4, bb):
        pltpu.make_async_copy(full_h.at[colbuf.at[b4]], inb.at[bb],
                              gsem.at[bb]).wait()

    def start_w(b4, bb):
        pltpu.async_copy(outb.at[bb], acc.at[rowbuf.at[b4]], ssem.at[bb],
                         add=True)

    def wait_w(bb):
        pltpu.make_async_copy(outb.at[bb], acc.at[pl.ds(0, CH)],
                              ssem.at[bb]).wait()

    def process(k, b4):
        bb = b4 % 2

        @pl.when(k + 1 < N_CHUNKS)
        def _():
            wait_idx(k + 1, (b4 + 1) % 4, 1 - bb)
            start_g((b4 + 1) % 4, 1 - bb)

        wait_g(b4, bb)

        @pl.when(k >= 2)
        def _():
            wait_w(bb)

        @pl.when(k + 2 < N_CHUNKS)
        def _():
            start_idx(k + 2, (b4 + 2) % 4, bb)

        @plsc.parallel_loop(0, CH, unroll=4)
        def _scale(r):
            vb = plsc.load_gather(valbuf, [_splat(b4), _splat(r)])
            for j in range(DB):
                outb[bb, r, pl.ds(16 * j, 16)] = inb[bb, r, pl.ds(16 * j, 16)] * vb

        start_w(b4, bb)

    start_idx(0, 0, 0)
    start_idx(1, 1, 1)
    wait_idx(0, 0, 0)
    start_g(0, 0)

    @pl.loop(0, N_CHUNKS - 1, step=4)
    def _quad(i):
        for j in range(4):
            process(i + j, j)

    process(N_CHUNKS - 1, 0)
    wait_w(0)
    wait_w(1)
    plsc.subcore_barrier()

    def wo_chunk(k, carry):
        idx = s + k * NS

        @pl.when(idx < N_RCH)
        def _():
            base = idx * CH

            @pl.when(c == 0)
            def _w0():
                pltpu.sync_copy(acc.at[pl.ds(base, CH)],
                                p0_h.at[pl.ds(base, CH)])

            @pl.when(c == 1)
            def _w1():
                pltpu.sync_copy(acc.at[pl.ds(base, CH)],
                                p1_h.at[pl.ds(base, CH)])

        return carry

    lax.fori_loop(0, N_RCH_PER_TILE, wo_chunk, 0)


_propagate = pl.kernel(
    _propagate_body,
    out_type=(
        jax.ShapeDtypeStruct((N_NODES, D), _f32),
        jax.ShapeDtypeStruct((N_NODES, D), _f32),
    ),
    mesh=_mesh,
    compiler_params=pltpu.CompilerParams(needs_layout_passes=False),
    scratch_types=[
        pltpu.VMEM_SHARED((N_NODES, D), _f32),
        pltpu.VMEM((4, CH), _i32),
        pltpu.VMEM((4, CH), _i32),
        pltpu.VMEM((4, CH), _f32),
        pltpu.VMEM((2, CH, D), _f32),
        pltpu.VMEM((2, CH, D), _f32),
        pltpu.SemaphoreType.DMA((2,)),
        pltpu.SemaphoreType.DMA((2,)),
        pltpu.SemaphoreType.DMA((2,)),
    ],
)


def _readout_body(uidx_h, iidx_h, p0_h, p1_h, out_h,
                  uix, iix, bu0, bu1, bi0, bi1, pbuf, outv, gsem):
    c = lax.axis_index("c")
    s = lax.axis_index("s")
    obase = (c * NS + s) * B_PER_TILE

    pltpu.sync_copy(uidx_h.at[pl.ds(obase, B_PER_TILE)], uix)
    pltpu.sync_copy(iidx_h.at[pl.ds(obase, B_PER_TILE)], iix)

    @plsc.parallel_loop(0, B_PER_TILE // 16, unroll=4)
    def _shift(k):
        iix[pl.ds(k * 16, 16)] = iix[pl.ds(k * 16, 16)] + NUM_USERS

    def start_gathers(cc, b):
        ui = uix.at[pl.ds(cc * BCH, BCH)]
        ii = iix.at[pl.ds(cc * BCH, BCH)]
        pltpu.async_copy(p0_h.at[ui], bu0.at[b], gsem.at[b])
        pltpu.async_copy(p1_h.at[ui], bu1.at[b], gsem.at[b])
        pltpu.async_copy(p0_h.at[ii], bi0.at[b], gsem.at[b])
        pltpu.async_copy(p1_h.at[ii], bi1.at[b], gsem.at[b])

    def wait_gathers(cc, b):
        ui = uix.at[pl.ds(cc * BCH, BCH)]
        ii = iix.at[pl.ds(cc * BCH, BCH)]
        pltpu.make_async_copy(p0_h.at[ui], bu0.at[b], gsem.at[b]).wait()
        pltpu.make_async_copy(p1_h.at[ui], bu1.at[b], gsem.at[b]).wait()
        pltpu.make_async_copy(p0_h.at[ii], bi0.at[b], gsem.at[b]).wait()
        pltpu.make_async_copy(p1_h.at[ii], bi1.at[b], gsem.at[b]).wait()

    def process(cc, b):
        wait_gathers(cc, b)

        def group(g, carry):
            def _el(e, carry2):
                r = g * 16 + e
                p = jnp.zeros((16,), _f32)
                for j in range(DB):
                    u = (bu0[b, r, pl.ds(16 * j, 16)]
                         + bu1[b, r, pl.ds(16 * j, 16)])
                    v = (bi0[b, r, pl.ds(16 * j, 16)]
                         + bi1[b, r, pl.ds(16 * j, 16)])
                    p = p + u * v
                pbuf[e, pl.ds(0, 16)] = p
                return carry2

            lax.fori_loop(0, 16, _el, 0)

            bvec = lax.iota(_i32, 16)

            def _red(j, a):
                return a + plsc.load_gather(pbuf, [bvec, _splat(j)])

            acc16 = lax.fori_loop(0, 16, _red, jnp.zeros((16,), _f32))
            outv[pl.ds(cc * BCH + g * 16, 16)] = acc16 * 0.25
            return carry

        lax.fori_loop(0, BCH // 16, group, 0)

        @pl.when(cc + 2 < NBCH)
        def _():
            start_gathers(cc + 2, b)

    start_gathers(0, 0)
    start_gathers(1, 1)

    @pl.loop(0, NBCH, step=2)
    def _pair(i):
        process(i, 0)
        process(i + 1, 1)

    pltpu.sync_copy(outv, out_h.at[pl.ds(obase, B_PER_TILE)])


_readout = pl.kernel(
    _readout_body,
    out_type=jax.ShapeDtypeStruct((B,), _f32),
    mesh=_mesh,
    compiler_params=pltpu.CompilerParams(needs_layout_passes=False),
    scratch_types=[
        pltpu.VMEM((B_PER_TILE,), _i32),
        pltpu.VMEM((B_PER_TILE,), _i32),
        pltpu.VMEM((2, BCH, D), _f32),
        pltpu.VMEM((2, BCH, D), _f32),
        pltpu.VMEM((2, BCH, D), _f32),
        pltpu.VMEM((2, BCH, D), _f32),
        pltpu.VMEM((16, 16), _f32),
        pltpu.VMEM((B_PER_TILE,), _f32),
        pltpu.SemaphoreType.DMA((2,)),
    ],
)


@jax.jit
def kernel(user_indices, item_indices, user_emb, item_emb, adj_row, adj_col, adj_val):
    full = jnp.concatenate([user_emb, item_emb], axis=0)
    p0, p1 = _propagate(full, adj_row, adj_col, adj_val)
    return _readout(user_indices, item_indices, p0, p1)


# speedup vs baseline: 1.0047x; 1.0047x over previous
import jax
import jax.numpy as jnp
from jax import lax
from jax.experimental import pallas as pl
from jax.experimental.pallas import tpu as pltpu
from jax.experimental.pallas import tpu_sc as plsc

NUM_USERS = 5000
N_NODES = 10000
D = 128
DB = D // 16

NC = 2
NS = 16
NW = NC * NS

E_TOTAL = 320000
CH = 80
EDGES_PER_TILE = E_TOTAL // NW
N_CHUNKS = EDGES_PER_TILE // CH

N_RCH = N_NODES // CH
N_RCH_PER_TILE = (N_RCH + NS - 1) // NS

B = 16384
B_PER_TILE = B // NW
BCH = 64
NBCH = B_PER_TILE // BCH

_mesh = plsc.VectorSubcoreMesh(
    core_axis_name="c", subcore_axis_name="s", num_cores=NC, num_subcores=NS
)

_f32 = jnp.float32
_i32 = jnp.int32


def _splat(x):
    return jnp.full((16,), x, _i32)


def _propagate_body(full_h, row_h, col_h, val_h, p0_h, p1_h,
                    acc, colbuf, rowbuf, valbuf, inb, outb,
                    lsem, gsem, ssem):
    c = lax.axis_index("c")
    s = lax.axis_index("s")
    wid = c * NS + s
    ebase = wid * EDGES_PER_TILE

    @pl.when(c == 1)
    def _zero_staging():
        @plsc.parallel_loop(0, CH, unroll=4)
        def _z(r):
            for j in range(DB):
                inb[0, r, pl.ds(16 * j, 16)] = jnp.zeros((16,), _f32)

    def init_chunk(k, carry):
        idx = s + k * NS

        @pl.when(idx < N_RCH)
        def _():
            base = idx * CH

            @pl.when(c == 0)
            def _c0():
                pltpu.sync_copy(full_h.at[pl.ds(base, CH)],
                                acc.at[pl.ds(base, CH)])

            @pl.when(c == 1)
            def _c1():
                pltpu.sync_copy(inb.at[0], acc.at[pl.ds(base, CH)])

        return carry

    lax.fori_loop(0, N_RCH_PER_TILE, init_chunk, 0)
    plsc.subcore_barrier()


    def edge_slice(k):
        return pl.ds(ebase + k * CH, CH)

    def start_idx(k, b4, bb):
        pltpu.async_copy(col_h.at[edge_slice(k)], colbuf.at[b4], lsem.at[bb])
        pltpu.async_copy(row_h.at[edge_slice(k)], rowbuf.at[b4], lsem.at[bb])
        pltpu.async_copy(val_h.at[edge_slice(k)], valbuf.at[b4], lsem.at[bb])

    def wait_idx(k, b4, bb):
        pltpu.make_async_copy(col_h.at[edge_slice(k)], colbuf.at[b4],
                              lsem.at[bb]).wait()
        pltpu.make_async_copy(row_h.at[edge_slice(k)], rowbuf.at[b4],
                              lsem.at[bb]).wait()
        pltpu.make_async_copy(val_h.at[edge_slice(k)], valbuf.at[b4],
                              lsem.at[bb]).wait()

    def start_g(b4, bb):
        pltpu.async_copy(full_h.at[colbuf.at[b4]], inb.at[bb], gsem.at[bb])

    def wait_g(b4, bb):
        pltpu.make_async_copy(full_h.at[colbuf.at[b4]], inb.at[bb],
                              gsem.at[bb]).wait()

    def start_w(b4, bb):
        pltpu.async_copy(outb.at[bb], acc.at[rowbuf.at[b4]], ssem.at[bb],
                         add=True)

    def wait_w(bb):
        pltpu.make_async_copy(outb.at[bb], acc.at[pl.ds(0, CH)],
                              ssem.at[bb]).wait()

    def process(k, b4):
        bb = b4 % 2

        @pl.when(k + 1 < N_CHUNKS)
        def _():
            wait_idx(k + 1, (b4 + 1) % 4, 1 - bb)
            start_g((b4 + 1) % 4, 1 - bb)

        wait_g(b4, bb)

        @pl.when(k >= 2)
        def _():
            wait_w(bb)

        @pl.when(k + 2 < N_CHUNKS)
        def _():
            start_idx(k + 2, (b4 + 2) % 4, bb)

        @plsc.parallel_loop(0, CH, unroll=8)
        def _scale(r):
            vb = plsc.load_gather(valbuf, [_splat(b4), _splat(r)])
            for j in range(DB):
                outb[bb, r, pl.ds(16 * j, 16)] = inb[bb, r, pl.ds(16 * j, 16)] * vb

        start_w(b4, bb)

    start_idx(0, 0, 0)
    start_idx(1, 1, 1)
    wait_idx(0, 0, 0)
    start_g(0, 0)

    @pl.loop(0, N_CHUNKS - 1, step=4)
    def _quad(i):
        for j in range(4):
            process(i + j, j)

    process(N_CHUNKS - 1, 0)
    wait_w(0)
    wait_w(1)
    plsc.subcore_barrier()

    def wo_chunk(k, carry):
        idx = s + k * NS

        @pl.when(idx < N_RCH)
        def _():
            base = idx * CH

            @pl.when(c == 0)
            def _w0():
                pltpu.sync_copy(acc.at[pl.ds(base, CH)],
                                p0_h.at[pl.ds(base, CH)])

            @pl.when(c == 1)
            def _w1():
                pltpu.sync_copy(acc.at[pl.ds(base, CH)],
                                p1_h.at[pl.ds(base, CH)])

        return carry

    lax.fori_loop(0, N_RCH_PER_TILE, wo_chunk, 0)


_propagate = pl.kernel(
    _propagate_body,
    out_type=(
        jax.ShapeDtypeStruct((N_NODES, D), _f32),
        jax.ShapeDtypeStruct((N_NODES, D), _f32),
    ),
    mesh=_mesh,
    compiler_params=pltpu.CompilerParams(needs_layout_passes=False),
    scratch_types=[
        pltpu.VMEM_SHARED((N_NODES, D), _f32),
        pltpu.VMEM((4, CH), _i32),
        pltpu.VMEM((4, CH), _i32),
        pltpu.VMEM((4, CH), _f32),
        pltpu.VMEM((2, CH, D), _f32),
        pltpu.VMEM((2, CH, D), _f32),
        pltpu.SemaphoreType.DMA((2,)),
        pltpu.SemaphoreType.DMA((2,)),
        pltpu.SemaphoreType.DMA((2,)),
    ],
)


def _readout_body(uidx_h, iidx_h, p0_h, p1_h, out_h,
                  uix, iix, bu0, bu1, bi0, bi1, pbuf, outv, gsem):
    c = lax.axis_index("c")
    s = lax.axis_index("s")
    obase = (c * NS + s) * B_PER_TILE

    pltpu.sync_copy(uidx_h.at[pl.ds(obase, B_PER_TILE)], uix)
    pltpu.sync_copy(iidx_h.at[pl.ds(obase, B_PER_TILE)], iix)

    @plsc.parallel_loop(0, B_PER_TILE // 16, unroll=4)
    def _shift(k):
        iix[pl.ds(k * 16, 16)] = iix[pl.ds(k * 16, 16)] + NUM_USERS

    def start_gathers(cc, b):
        ui = uix.at[pl.ds(cc * BCH, BCH)]
        ii = iix.at[pl.ds(cc * BCH, BCH)]
        pltpu.async_copy(p0_h.at[ui], bu0.at[b], gsem.at[b])
        pltpu.async_copy(p1_h.at[ui], bu1.at[b], gsem.at[b])
        pltpu.async_copy(p0_h.at[ii], bi0.at[b], gsem.at[b])
        pltpu.async_copy(p1_h.at[ii], bi1.at[b], gsem.at[b])

    def wait_gathers(cc, b):
        ui = uix.at[pl.ds(cc * BCH, BCH)]
        ii = iix.at[pl.ds(cc * BCH, BCH)]
        pltpu.make_async_copy(p0_h.at[ui], bu0.at[b], gsem.at[b]).wait()
        pltpu.make_async_copy(p1_h.at[ui], bu1.at[b], gsem.at[b]).wait()
        pltpu.make_async_copy(p0_h.at[ii], bi0.at[b], gsem.at[b]).wait()
        pltpu.make_async_copy(p1_h.at[ii], bi1.at[b], gsem.at[b]).wait()

    def process(cc, b):
        wait_gathers(cc, b)

        def group(g, carry):
            @plsc.parallel_loop(0, 16, unroll=2)
            def _el(e):
                r = g * 16 + e
                p = jnp.zeros((16,), _f32)
                for j in range(DB):
                    u = (bu0[b, r, pl.ds(16 * j, 16)]
                         + bu1[b, r, pl.ds(16 * j, 16)])
                    v = (bi0[b, r, pl.ds(16 * j, 16)]
                         + bi1[b, r, pl.ds(16 * j, 16)])
                    p = p + u * v
                pbuf[e, pl.ds(0, 16)] = p

            bvec = lax.iota(_i32, 16)

            def _red(j, a):
                return a + plsc.load_gather(pbuf, [bvec, _splat(j)])

            acc16 = lax.fori_loop(0, 16, _red, jnp.zeros((16,), _f32))
            outv[pl.ds(cc * BCH + g * 16, 16)] = acc16 * 0.25
            return carry

        lax.fori_loop(0, BCH // 16, group, 0)

        @pl.when(cc + 2 < NBCH)
        def _():
            start_gathers(cc + 2, b)

    start_gathers(0, 0)
    start_gathers(1, 1)

    @pl.loop(0, NBCH, step=2)
    def _pair(i):
        process(i, 0)
        process(i + 1, 1)

    pltpu.sync_copy(outv, out_h.at[pl.ds(obase, B_PER_TILE)])


_readout = pl.kernel(
    _readout_body,
    out_type=jax.ShapeDtypeStruct((B,), _f32),
    mesh=_mesh,
    compiler_params=pltpu.CompilerParams(needs_layout_passes=False),
    scratch_types=[
        pltpu.VMEM((B_PER_TILE,), _i32),
        pltpu.VMEM((B_PER_TILE,), _i32),
        pltpu.VMEM((2, BCH, D), _f32),
        pltpu.VMEM((2, BCH, D), _f32),
        pltpu.VMEM((2, BCH, D), _f32),
        pltpu.VMEM((2, BCH, D), _f32),
        pltpu.VMEM((16, 16), _f32),
        pltpu.VMEM((B_PER_TILE,), _f32),
        pltpu.SemaphoreType.DMA((2,)),
    ],
)


@jax.jit
def kernel(user_indices, item_indices, user_emb, item_emb, adj_row, adj_col, adj_val):
    full = jnp.concatenate([user_emb, item_emb], axis=0)
    p0, p1 = _propagate(full, adj_row, adj_col, adj_val)
    return _readout(user_indices, item_indices, p0, p1)


# speedup vs baseline: 1.0178x; 1.0130x over previous
import jax
import jax.numpy as jnp
from jax import lax
from jax.experimental import pallas as pl
from jax.experimental.pallas import tpu as pltpu
from jax.experimental.pallas import tpu_sc as plsc

NUM_USERS = 5000
N_NODES = 10000
D = 128
DB = D // 16

NC = 2
NS = 16
NW = NC * NS

E_TOTAL = 320000
CH = 80
EDGES_PER_TILE = E_TOTAL // NW
N_CHUNKS = EDGES_PER_TILE // CH

N_RCH = N_NODES // CH
N_RCH_PER_TILE = (N_RCH + NS - 1) // NS

B = 16384
B_PER_TILE = B // NW
BCH = 64
NBCH = B_PER_TILE // BCH

_mesh = plsc.VectorSubcoreMesh(
    core_axis_name="c", subcore_axis_name="s", num_cores=NC, num_subcores=NS
)

_f32 = jnp.float32
_i32 = jnp.int32


def _splat(x):
    return jnp.full((16,), x, _i32)


def _propagate_body(full_h, row_h, col_h, val_h, p0_h, p1_h,
                    acc, colbuf, rowbuf, valbuf, inb, outb,
                    lsem, gsem, ssem, isem):
    c = lax.axis_index("c")
    s = lax.axis_index("s")
    wid = c * NS + s
    ebase = wid * EDGES_PER_TILE

    @pl.when(c == 1)
    def _zero_staging():
        @plsc.parallel_loop(0, CH, unroll=4)
        def _z(r):
            for j in range(DB):
                inb[0, r, pl.ds(16 * j, 16)] = jnp.zeros((16,), _f32)

    def init_issue(k):
        idx = s + k * NS

        @pl.when(idx < N_RCH)
        def _():
            base = idx * CH

            @pl.when(c == 0)
            def _c0():
                pltpu.async_copy(full_h.at[pl.ds(base, CH)],
                                 acc.at[pl.ds(base, CH)], isem)

            @pl.when(c == 1)
            def _c1():
                pltpu.async_copy(inb.at[0], acc.at[pl.ds(base, CH)], isem)

    def init_wait(k):
        idx = s + k * NS

        @pl.when(idx < N_RCH)
        def _():
            base = idx * CH

            @pl.when(c == 0)
            def _c0():
                pltpu.make_async_copy(full_h.at[pl.ds(base, CH)],
                                      acc.at[pl.ds(base, CH)], isem).wait()

            @pl.when(c == 1)
            def _c1():
                pltpu.make_async_copy(inb.at[0], acc.at[pl.ds(base, CH)],
                                      isem).wait()

    init_issue(0)

    def init_step(k, carry):
        init_issue(k)
        init_wait(k - 1)
        return carry

    lax.fori_loop(1, N_RCH_PER_TILE, init_step, 0)
    init_wait(N_RCH_PER_TILE - 1)
    plsc.subcore_barrier()


    def edge_slice(k):
        return pl.ds(ebase + k * CH, CH)

    def start_idx(k, b4, bb):
        pltpu.async_copy(col_h.at[edge_slice(k)], colbuf.at[b4], lsem.at[bb])
        pltpu.async_copy(row_h.at[edge_slice(k)], rowbuf.at[b4], lsem.at[bb])
        pltpu.async_copy(val_h.at[edge_slice(k)], valbuf.at[b4], lsem.at[bb])

    def wait_idx(k, b4, bb):
        pltpu.make_async_copy(col_h.at[edge_slice(k)], colbuf.at[b4],
                              lsem.at[bb]).wait()
        pltpu.make_async_copy(row_h.at[edge_slice(k)], rowbuf.at[b4],
                              lsem.at[bb]).wait()
        pltpu.make_async_copy(val_h.at[edge_slice(k)], valbuf.at[b4],
                              lsem.at[bb]).wait()

    def start_g(b4, bb):
        pltpu.async_copy(full_h.at[colbuf.at[b4]], inb.at[bb], gsem.at[bb])

    def wait_g(b4, bb):
        pltpu.make_async_copy(full_h.at[colbuf.at[b4]], inb.at[bb],
                              gsem.at[bb]).wait()

    def start_w(b4, bb):
        pltpu.async_copy(outb.at[bb], acc.at[rowbuf.at[b4]], ssem.at[bb],
                         add=True)

    def wait_w(bb):
        pltpu.make_async_copy(outb.at[bb], acc.at[pl.ds(0, CH)],
                              ssem.at[bb]).wait()

    def process(k, b4):
        bb = b4 % 2

        @pl.when(k + 1 < N_CHUNKS)
        def _():
            wait_idx(k + 1, (b4 + 1) % 4, 1 - bb)
            start_g((b4 + 1) % 4, 1 - bb)

        wait_g(b4, bb)

        @pl.when(k >= 2)
        def _():
            wait_w(bb)

        @pl.when(k + 2 < N_CHUNKS)
        def _():
            start_idx(k + 2, (b4 + 2) % 4, bb)

        @plsc.parallel_loop(0, CH, unroll=4)
        def _scale(r):
            vb = plsc.load_gather(valbuf, [_splat(b4), _splat(r)])
            for j in range(DB):
                outb[bb, r, pl.ds(16 * j, 16)] = inb[bb, r, pl.ds(16 * j, 16)] * vb

        start_w(b4, bb)

    start_idx(0, 0, 0)
    start_idx(1, 1, 1)
    wait_idx(0, 0, 0)
    start_g(0, 0)

    @pl.loop(0, N_CHUNKS - 1, step=4)
    def _quad(i):
        for j in range(4):
            process(i + j, j)

    process(N_CHUNKS - 1, 0)
    wait_w(0)
    wait_w(1)
    plsc.subcore_barrier()

    def wo_issue(k):
        idx = s + k * NS

        @pl.when(idx < N_RCH)
        def _():
            base = idx * CH

            @pl.when(c == 0)
            def _w0():
                pltpu.async_copy(acc.at[pl.ds(base, CH)],
                                 p0_h.at[pl.ds(base, CH)], isem)

            @pl.when(c == 1)
            def _w1():
                pltpu.async_copy(acc.at[pl.ds(base, CH)],
                                 p1_h.at[pl.ds(base, CH)], isem)

    def wo_wait(k):
        idx = s + k * NS

        @pl.when(idx < N_RCH)
        def _():
            base = idx * CH

            @pl.when(c == 0)
            def _w0():
                pltpu.make_async_copy(acc.at[pl.ds(base, CH)],
                                      p0_h.at[pl.ds(base, CH)], isem).wait()

            @pl.when(c == 1)
            def _w1():
                pltpu.make_async_copy(acc.at[pl.ds(base, CH)],
                                      p1_h.at[pl.ds(base, CH)], isem).wait()

    wo_issue(0)

    def wo_step(k, carry):
        wo_issue(k)
        wo_wait(k - 1)
        return carry

    lax.fori_loop(1, N_RCH_PER_TILE, wo_step, 0)
    wo_wait(N_RCH_PER_TILE - 1)


_propagate = pl.kernel(
    _propagate_body,
    out_type=(
        jax.ShapeDtypeStruct((N_NODES, D), _f32),
        jax.ShapeDtypeStruct((N_NODES, D), _f32),
    ),
    mesh=_mesh,
    compiler_params=pltpu.CompilerParams(needs_layout_passes=False),
    scratch_types=[
        pltpu.VMEM_SHARED((N_NODES, D), _f32),
        pltpu.VMEM((4, CH), _i32),
        pltpu.VMEM((4, CH), _i32),
        pltpu.VMEM((4, CH), _f32),
        pltpu.VMEM((2, CH, D), _f32),
        pltpu.VMEM((2, CH, D), _f32),
        pltpu.SemaphoreType.DMA((2,)),
        pltpu.SemaphoreType.DMA((2,)),
        pltpu.SemaphoreType.DMA((2,)),
        pltpu.SemaphoreType.DMA,
    ],
)


def _readout_body(uidx_h, iidx_h, p0_h, p1_h, out_h,
                  uix, iix, bu0, bu1, bi0, bi1, pbuf, outv, gsem):
    c = lax.axis_index("c")
    s = lax.axis_index("s")
    obase = (c * NS + s) * B_PER_TILE

    pltpu.sync_copy(uidx_h.at[pl.ds(obase, B_PER_TILE)], uix)
    pltpu.sync_copy(iidx_h.at[pl.ds(obase, B_PER_TILE)], iix)

    @plsc.parallel_loop(0, B_PER_TILE // 16, unroll=4)
    def _shift(k):
        iix[pl.ds(k * 16, 16)] = iix[pl.ds(k * 16, 16)] + NUM_USERS

    def start_gathers(cc, b):
        ui = uix.at[pl.ds(cc * BCH, BCH)]
        ii = iix.at[pl.ds(cc * BCH, BCH)]
        pltpu.async_copy(p0_h.at[ui], bu0.at[b], gsem.at[b])
        pltpu.async_copy(p1_h.at[ui], bu1.at[b], gsem.at[b])
        pltpu.async_copy(p0_h.at[ii], bi0.at[b], gsem.at[b])
        pltpu.async_copy(p1_h.at[ii], bi1.at[b], gsem.at[b])

    def wait_gathers(cc, b):
        ui = uix.at[pl.ds(cc * BCH, BCH)]
        ii = iix.at[pl.ds(cc * BCH, BCH)]
        pltpu.make_async_copy(p0_h.at[ui], bu0.at[b], gsem.at[b]).wait()
        pltpu.make_async_copy(p1_h.at[ui], bu1.at[b], gsem.at[b]).wait()
        pltpu.make_async_copy(p0_h.at[ii], bi0.at[b], gsem.at[b]).wait()
        pltpu.make_async_copy(p1_h.at[ii], bi1.at[b], gsem.at[b]).wait()

    def process(cc, b):
        wait_gathers(cc, b)

        def group(g, carry):
            def _el(e, carry2):
                r = g * 16 + e
                p = jnp.zeros((16,), _f32)
                for j in range(DB):
                    u = (bu0[b, r, pl.ds(16 * j, 16)]
                         + bu1[b, r, pl.ds(16 * j, 16)])
                    v = (bi0[b, r, pl.ds(16 * j, 16)]
                         + bi1[b, r, pl.ds(16 * j, 16)])
                    p = p + u * v
                pbuf[e, pl.ds(0, 16)] = p
                return carry2

            lax.fori_loop(0, 16, _el, 0)

            bvec = lax.iota(_i32, 16)

            def _red(j, a):
                return a + plsc.load_gather(pbuf, [bvec, _splat(j)])

            acc16 = lax.fori_loop(0, 16, _red, jnp.zeros((16,), _f32))
            outv[pl.ds(cc * BCH + g * 16, 16)] = acc16 * 0.25
            return carry

        lax.fori_loop(0, BCH // 16, group, 0)

        @pl.when(cc + 2 < NBCH)
        def _():
            start_gathers(cc + 2, b)

    start_gathers(0, 0)
    start_gathers(1, 1)

    @pl.loop(0, NBCH, step=2)
    def _pair(i):
        process(i, 0)
        process(i + 1, 1)

    pltpu.sync_copy(outv, out_h.at[pl.ds(obase, B_PER_TILE)])


_readout = pl.kernel(
    _readout_body,
    out_type=jax.ShapeDtypeStruct((B,), _f32),
    mesh=_mesh,
    compiler_params=pltpu.CompilerParams(needs_layout_passes=False),
    scratch_types=[
        pltpu.VMEM((B_PER_TILE,), _i32),
        pltpu.VMEM((B_PER_TILE,), _i32),
        pltpu.VMEM((2, BCH, D), _f32),
        pltpu.VMEM((2, BCH, D), _f32),
        pltpu.VMEM((2, BCH, D), _f32),
        pltpu.VMEM((2, BCH, D), _f32),
        pltpu.VMEM((16, 16), _f32),
        pltpu.VMEM((B_PER_TILE,), _f32),
        pltpu.SemaphoreType.DMA((2,)),
    ],
)


@jax.jit
def kernel(user_indices, item_indices, user_emb, item_emb, adj_row, adj_col, adj_val):
    full = jnp.concatenate([user_emb, item_emb], axis=0)
    p0, p1 = _propagate(full, adj_row, adj_col, adj_val)
    return _readout(user_indices, item_indices, p0, p1)
